# Initial kernel scaffold; baseline (speedup 1.0000x reference)
import functools

import jax
import jax.numpy as jnp
from jax import lax
from jax.experimental import pallas as pl
from jax.experimental.pallas import tpu as pltpu
from jax.experimental.pallas import tpu_sc as plsc

N = 10000
E = 320000
D = 128
K1 = 8000
K2 = 6400
OUT_DIM = 40000
NP = 10240

NC = 2
NS = 16
NW = NC * NS
EPT = E // NW
CHUNK = 80
NCHUNK = EPT // CHUNK
RPT = N // NS

_MIN32 = jnp.int32(-(2 ** 31))


def _f32_sort_key(bits):
    return jnp.where(bits < 0, (~bits) ^ _MIN32, bits)


def _sc_mesh():
    return plsc.VectorSubcoreMesh(
        core_axis_name="c", subcore_axis_name="s", num_cores=NC, num_subcores=NS)


def _deg_body(mask_hbm, src_hbm, dst_hbm, out_hbm, mask_v, deg_v, src_v, dst_v):
    c = lax.axis_index("c")
    s = lax.axis_index("s")
    wid = c * NS + s
    base = wid * EPT
    pltpu.sync_copy(mask_hbm, mask_v)
    pltpu.sync_copy(src_hbm.at[pl.ds(base, EPT)], src_v)
    pltpu.sync_copy(dst_hbm.at[pl.ds(base, EPT)], dst_v)

    def zero(i, carry):
        deg_v[pl.ds(i * 16, 16)] = jnp.zeros((16,), jnp.float32)
        return carry

    lax.fori_loop(0, N // 16, zero, 0)

    def upd(i, carry):
        si = src_v[pl.ds(i * 16, 16)]
        di = dst_v[pl.ds(i * 16, 16)]
        v = plsc.load_gather(mask_v, [si])
        plsc.addupdate_scatter(deg_v, [di], v)
        return carry

    lax.fori_loop(0, EPT // 16, upd, 0)
    pltpu.sync_copy(deg_v, out_hbm.at[wid])


def _sc_deg(maskf, src, dst):
    f = pl.kernel(
        _deg_body,
        out_type=jax.ShapeDtypeStruct((NW, N), jnp.float32),
        mesh=_sc_mesh(),
        scratch_types=[
            pltpu.VMEM((N,), jnp.float32),
            pltpu.VMEM((N,), jnp.float32),
            pltpu.VMEM((EPT,), jnp.int32),
            pltpu.VMEM((EPT,), jnp.int32),
        ],
    )
    return f(maskf, src, dst)


def _agg_body(u_hbm, src_hbm, dst_hbm, out_hbm, sidx_v, didx_v, rows_v, shared, sem):
    c = lax.axis_index("c")
    s = lax.axis_index("s")
    wid = c * NS + s
    base = wid * EPT

    def zrow(i, carry):
        def zcol(j, carry2):
            rows_v[i, pl.ds(j * 16, 16)] = jnp.zeros((16,), jnp.float32)
            return carry2
        return lax.fori_loop(0, D // 16, zcol, carry)

    lax.fori_loop(0, CHUNK, zrow, 0)
    off = s * RPT
    for t in range(RPT // CHUNK):
        pltpu.sync_copy(rows_v, shared.at[pl.ds(off + t * CHUNK, CHUNK)])
    rem = RPT - (RPT // CHUNK) * CHUNK
    pltpu.sync_copy(rows_v.at[pl.ds(0, rem)],
                    shared.at[pl.ds(off + (RPT // CHUNK) * CHUNK, rem)])
    plsc.subcore_barrier()

    def chunk(i, carry):
        cb = base + i * CHUNK
        pltpu.sync_copy(src_hbm.at[pl.ds(cb, CHUNK)], sidx_v)
        pltpu.sync_copy(dst_hbm.at[pl.ds(cb, CHUNK)], didx_v)
        pltpu.async_copy(u_hbm.at[sidx_v], rows_v, sem).wait()
        pltpu.sync_copy(rows_v, shared.at[didx_v], add=True)
        return carry

    lax.fori_loop(0, NCHUNK, chunk, 0)
    plsc.subcore_barrier()
    pltpu.sync_copy(shared.at[pl.ds(off, RPT)],
                    out_hbm.at[pl.ds(c * N + off, RPT)])


def _sc_agg(u, src, dst):
    f = pl.kernel(
        _agg_body,
        out_type=jax.ShapeDtypeStruct((2 * N, D), jnp.float32),
        mesh=_sc_mesh(),
        scratch_types=[
            pltpu.VMEM((CHUNK,), jnp.int32),
            pltpu.VMEM((CHUNK,), jnp.int32),
            pltpu.VMEM((CHUNK, D), jnp.float32),
            pltpu.VMEM_SHARED((N, D), jnp.float32),
            pltpu.SemaphoreType.DMA,
        ],
    )
    return f(u, src, dst)


_RB = 1024
_GRID_R = (N + _RB - 1) // _RB


def _scale_body(x_ref, rs_ref, w_ref, deg_ref, u_ref, dis_ref):
    xw = jnp.dot(x_ref[...] * rs_ref[...], w_ref[...],
                 preferred_element_type=jnp.float32)
    deg = jnp.sum(deg_ref[...], axis=1, keepdims=True) + 1.0
    dis = lax.rsqrt(jnp.maximum(deg, 1.0))
    u_ref[...] = xw * dis
    dis_ref[...] = dis


def _tc_scale(x, rs, w, deg_t):
    return pl.pallas_call(
        _scale_body,
        grid=(_GRID_R,),
        in_specs=[
            pl.BlockSpec((_RB, D), lambda i: (i, 0)),
            pl.BlockSpec((_RB, 1), lambda i: (i, 0)),
            pl.BlockSpec((D, D), lambda i: (0, 0)),
            pl.BlockSpec((_RB, NW), lambda i: (i, 0)),
        ],
        out_specs=[
            pl.BlockSpec((_RB, D), lambda i: (i, 0)),
            pl.BlockSpec((_RB, 1), lambda i: (i, 0)),
        ],
        out_shape=[
            jax.ShapeDtypeStruct((N, D), jnp.float32),
            jax.ShapeDtypeStruct((N, 1), jnp.float32),
        ],
    )(x, rs, w, deg_t)


def _post_body(a0_ref, a1_ref, u_ref, dis_ref, b_ref, p_ref, m_ref, h_ref, sc_ref):
    t = (a0_ref[...] + a1_ref[...] + u_ref[...]) * dis_ref[...] + b_ref[...][None, :]
    h = jnp.maximum(t, 0.0)
    h_ref[...] = h
    pv = p_ref[...]
    inv = lax.rsqrt(jnp.sum(pv * pv))
    s = jnp.sum(h * pv[None, :], axis=1, keepdims=True) * inv
    sc = jnp.tanh(s)
    sc_ref[...] = jnp.where(m_ref[...] > 0, sc, -2.0)


def _tc_post(a0, a1, u, dis, b, p, m):
    return pl.pallas_call(
        _post_body,
        grid=(_GRID_R,),
        in_specs=[
            pl.BlockSpec((_RB, D), lambda i: (i, 0)),
            pl.BlockSpec((_RB, D), lambda i: (i, 0)),
            pl.BlockSpec((_RB, D), lambda i: (i, 0)),
            pl.BlockSpec((_RB, 1), lambda i: (i, 0)),
            pl.BlockSpec((D,), lambda i: (0,)),
            pl.BlockSpec((D,), lambda i: (0,)),
            pl.BlockSpec((_RB, 1), lambda i: (i, 0)),
        ],
        out_specs=[
            pl.BlockSpec((_RB, D), lambda i: (i, 0)),
            pl.BlockSpec((_RB, 1), lambda i: (i, 0)),
        ],
        out_shape=[
            jax.ShapeDtypeStruct((N, D), jnp.float32),
            jax.ShapeDtypeStruct((N, 1), jnp.float32),
        ],
    )(a0, a1, u, dis, b, p, m)


_LO0 = -1082130433
_HI0 = 1069547520


def _sel_body(k, s_ref, m_ref, g_ref):
    s = s_ref[...]
    key = _f32_sort_key(lax.bitcast_convert_type(s, jnp.int32))

    def bis(_, lohi):
        lo, hi = lohi
        mid = (lo >> 1) + (hi >> 1) + (lo & hi & 1)
        cnt = jnp.sum((key >= mid).astype(jnp.int32))
        ok = cnt >= k
        return jnp.where(ok, mid, lo), jnp.where(ok, hi, mid)

    lo, _ = lax.fori_loop(0, 32, bis, (jnp.int32(_LO0), jnp.int32(_HI0)))
    thr = lo
    gt = key > thr
    cnt_gt = jnp.sum(gt.astype(jnp.int32))
    need = k - cnt_gt
    tie = key == thr
    idx = (lax.broadcasted_iota(jnp.int32, (NP // 128, 128), 0) * 128
           + lax.broadcasted_iota(jnp.int32, (NP // 128, 128), 1))

    def bis2(_, lohi):
        lo2, hi2 = lohi
        mid = (lo2 + hi2) >> 1
        cnt = jnp.sum((tie & (idx < mid)).astype(jnp.int32))
        ok = cnt >= need
        return jnp.where(ok, lo2, mid), jnp.where(ok, mid, hi2)

    _, cut = lax.fori_loop(0, 14, bis2, (jnp.int32(0), jnp.int32(NP)))
    m = (gt | (tie & (idx < cut))).astype(jnp.float32)
    m_ref[...] = m
    g_ref[...] = m * s


def _tc_select(sp, k):
    return pl.pallas_call(
        functools.partial(_sel_body, k),
        out_shape=[
            jax.ShapeDtypeStruct((NP // 128, 128), jnp.float32),
            jax.ShapeDtypeStruct((NP // 128, 128), jnp.float32),
        ],
    )(sp)


def _red_body(h_ref, g_ref, o_ref):
    i = pl.program_id(0)
    rb = i * _RB + lax.broadcasted_iota(jnp.int32, (_RB, 1), 0)
    contrib = jnp.where(rb < N, h_ref[...] * g_ref[...], 0.0)
    part = jnp.sum(contrib, axis=0, keepdims=True)

    @pl.when(i == 0)
    def _():
        o_ref[...] = part

    @pl.when(i > 0)
    def _():
        o_ref[...] = o_ref[...] + part


def _tc_reduce(h, g):
    return pl.pallas_call(
        _red_body,
        grid=(_GRID_R,),
        in_specs=[
            pl.BlockSpec((_RB, D), lambda i: (i, 0)),
            pl.BlockSpec((_RB, 1), lambda i: (i, 0)),
        ],
        out_specs=pl.BlockSpec((1, D), lambda i: (0, 0)),
        out_shape=jax.ShapeDtypeStruct((1, D), jnp.float32),
    )(h, g)


def _z_body(x1_ref, x2_ref, w1_ref, b1_ref, w2_ref, b2_ref, wv_ref, bv_ref,
            z_ref, v_ref):
    z0 = x1_ref[...] * (1.0 / K1) + x2_ref[...] * (1.0 / K2)
    z1 = jnp.dot(z0, w1_ref[...], preferred_element_type=jnp.float32) + b1_ref[...][None, :]
    z2 = jnp.dot(z1, w2_ref[...], preferred_element_type=jnp.float32) + b2_ref[...][None, :]
    z_ref[...] = z2
    v_ref[...] = (jnp.dot(z2, wv_ref[...], preferred_element_type=jnp.float32)
                  + bv_ref[...][None, :])


def _tc_z(xs1, xs2, w1, b1, w2, b2, wv, bv):
    return pl.pallas_call(
        _z_body,
        out_shape=[
            jax.ShapeDtypeStruct((1, 64), jnp.float32),
            jax.ShapeDtypeStruct((1, 1), jnp.float32),
        ],
    )(xs1, xs2, w1, b1, w2, b2, wv, bv)


_CB = 2048
_GRID_C = (OUT_DIM + _CB - 1) // _CB


def _out_body(z_ref, w_ref, b_ref, o_ref):
    o_ref[...] = jnp.tanh(
        jnp.dot(z_ref[...], w_ref[...], preferred_element_type=jnp.float32)
        + b_ref[...][None, :])


def _tc_out(z, w3, b3):
    return pl.pallas_call(
        _out_body,
        grid=(_GRID_C,),
        in_specs=[
            pl.BlockSpec((1, 64), lambda i: (0, 0)),
            pl.BlockSpec((64, _CB), lambda i: (0, i)),
            pl.BlockSpec((_CB,), lambda i: (i,)),
        ],
        out_specs=pl.BlockSpec((1, _CB), lambda i: (0, i)),
        out_shape=jax.ShapeDtypeStruct((1, OUT_DIM), jnp.float32),
    )(z, w3, b3)


def _pad_scores(score_col):
    s = jnp.reshape(score_col, (N,))
    s = jnp.concatenate([s, jnp.full((NP - N,), -3.0, jnp.float32)])
    return jnp.reshape(s, (NP // 128, 128))


def kernel(x, edge_index, W1, b1, p1, W2, b2, p2, lin1_W, lin1_b, lin2_W,
           lin2_b, lin3_W, lin3_b, linV_W, linV_b):
    src = edge_index[0]
    dst = edge_index[1]
    ones_n = jnp.ones((N,), jnp.float32)
    ones_c = jnp.ones((N, 1), jnp.float32)

    degp1 = _sc_deg(ones_n, src, dst)
    u1, dis1 = _tc_scale(x, ones_c, W1, degp1.T)
    aggf1 = _sc_agg(u1, src, dst)
    h, score1 = _tc_post(aggf1[:N], aggf1[N:], u1, dis1, b1, p1, ones_c)

    m1p, g1p = _tc_select(_pad_scores(score1), K1)
    mask1 = jnp.reshape(m1p, (NP,))[:N]
    g1 = jnp.reshape(g1p, (NP,))[:N, None]
    xs1 = _tc_reduce(h, g1)

    degp2 = _sc_deg(mask1, src, dst)
    u2, dis2 = _tc_scale(h, g1, W2, degp2.T)
    aggf2 = _sc_agg(u2, src, dst)
    h2, score2 = _tc_post(aggf2[:N], aggf2[N:], u2, dis2, b2, p2, mask1[:, None])

    _, g2p = _tc_select(_pad_scores(score2), K2)
    g2 = jnp.reshape(g2p, (NP,))[:N, None]
    xs2 = _tc_reduce(h2, g2)

    z, value = _tc_z(xs1, xs2, lin1_W, lin1_b, lin2_W, lin2_b, linV_W, linV_b)
    out = _tc_out(z, lin3_W, lin3_b)
    return (out, value)


# speedup vs baseline: 18.9457x; 18.9457x over previous
import functools

import jax
import jax.numpy as jnp
from jax import lax
from jax.experimental import pallas as pl
from jax.experimental.pallas import tpu as pltpu
from jax.experimental.pallas import tpu_sc as plsc

N = 10000
E = 320000
D = 128
K1 = 8000
K2 = 6400
OUT_DIM = 40000
NPAD = 10240

NC = 2
NS = 16
NW = NC * NS
EPT = E // NW
CHUNK = 80
NCHUNK = EPT // CHUNK
RPT = NPAD // NS
DW = 128


def _f32_sort_key(bits):
    return jnp.where(bits < 0, (~bits) ^ (-2147483648), bits)


def _sc_mesh():
    return plsc.VectorSubcoreMesh(
        core_axis_name="c", subcore_axis_name="s", num_cores=NC, num_subcores=NS)


def _agg_body(w, u_hbm, src_hbm, dst_hbm, out_hbm, sidx_v, didx_v, rows_v,
              shared, sem):
    c = lax.axis_index("c")
    s = lax.axis_index("s")
    wid = c * NS + s
    base = wid * EPT

    def zrow(i, carry):
        def zcol(j, carry2):
            rows_v[i, pl.ds(j * 16, 16)] = jnp.zeros((16,), jnp.float32)
            return carry2
        return lax.fori_loop(0, w // 16, zcol, carry)

    lax.fori_loop(0, CHUNK, zrow, 0)
    off = s * RPT
    for t in range(RPT // CHUNK):
        pltpu.sync_copy(rows_v, shared.at[pl.ds(off + t * CHUNK, CHUNK)])
    plsc.subcore_barrier()

    def chunk(i, carry):
        cb = base + i * CHUNK
        pltpu.sync_copy(src_hbm.at[pl.ds(cb, CHUNK)], sidx_v)
        pltpu.sync_copy(dst_hbm.at[pl.ds(cb, CHUNK)], didx_v)
        pltpu.async_copy(u_hbm.at[sidx_v], rows_v, sem).wait()
        pltpu.sync_copy(rows_v, shared.at[didx_v], add=True)
        return carry

    lax.fori_loop(0, NCHUNK, chunk, 0)
    plsc.subcore_barrier()
    pltpu.sync_copy(shared.at[pl.ds(off, RPT)],
                    out_hbm.at[pl.ds(c * NPAD + off, RPT)])


def _sc_agg(u, src, dst):
    w = u.shape[1]
    f = pl.kernel(
        functools.partial(_agg_body, w),
        out_type=jax.ShapeDtypeStruct((2 * NPAD, w), jnp.float32),
        mesh=_sc_mesh(),
        scratch_types=[
            pltpu.VMEM((CHUNK,), jnp.int32),
            pltpu.VMEM((CHUNK,), jnp.int32),
            pltpu.VMEM((CHUNK, w), jnp.float32),
            pltpu.VMEM_SHARED((NPAD, w), jnp.float32),
            pltpu.SemaphoreType.DMA,
        ],
    )
    return f(u, src, dst)


def _sc_deg(mask_pad, src, dst):
    table = jnp.broadcast_to(mask_pad[:, None], (NPAD, DW))
    return _sc_agg(table, src, dst)


_RB = 1024
_GRID_R = NPAD // _RB


def _scale_body(x_ref, rs_ref, w_ref, d0_ref, d1_ref, u_ref, dis_ref):
    xw = jnp.dot(x_ref[...] * rs_ref[...], w_ref[...],
                 preferred_element_type=jnp.float32)
    deg = d0_ref[...][:, 0:1] + d1_ref[...][:, 0:1] + 1.0
    dis = lax.rsqrt(jnp.maximum(deg, 1.0))
    u_ref[...] = xw * dis
    dis_ref[...] = dis


def _tc_scale(x, rs, w, degp):
    return pl.pallas_call(
        _scale_body,
        grid=(_GRID_R,),
        in_specs=[
            pl.BlockSpec((_RB, D), lambda i: (i, 0)),
            pl.BlockSpec((_RB, 1), lambda i: (i, 0)),
            pl.BlockSpec((D, D), lambda i: (0, 0)),
            pl.BlockSpec((_RB, DW), lambda i: (i, 0)),
            pl.BlockSpec((_RB, DW), lambda i: (i + _GRID_R, 0)),
        ],
        out_specs=[
            pl.BlockSpec((_RB, D), lambda i: (i, 0)),
            pl.BlockSpec((_RB, 1), lambda i: (i, 0)),
        ],
        out_shape=[
            jax.ShapeDtypeStruct((NPAD, D), jnp.float32),
            jax.ShapeDtypeStruct((NPAD, 1), jnp.float32),
        ],
    )(x, rs, w, degp, degp)


def _post_body(a0_ref, a1_ref, u_ref, dis_ref, b_ref, p_ref, m_ref, h_ref, sc_ref):
    t = (a0_ref[...] + a1_ref[...] + u_ref[...]) * dis_ref[...] + b_ref[...][None, :]
    h = jnp.maximum(t, 0.0)
    h_ref[...] = h
    pv = p_ref[...]
    inv = lax.rsqrt(jnp.sum(pv * pv))
    s = jnp.sum(h * pv[None, :], axis=1, keepdims=True) * inv
    sc = jnp.tanh(s)
    sc_ref[...] = jnp.where(m_ref[...] > 0, sc, -2.0)


def _tc_post(aggf, u, dis, b, p, m):
    return pl.pallas_call(
        _post_body,
        grid=(_GRID_R,),
        in_specs=[
            pl.BlockSpec((_RB, D), lambda i: (i, 0)),
            pl.BlockSpec((_RB, D), lambda i: (i + _GRID_R, 0)),
            pl.BlockSpec((_RB, D), lambda i: (i, 0)),
            pl.BlockSpec((_RB, 1), lambda i: (i, 0)),
            pl.BlockSpec((D,), lambda i: (0,)),
            pl.BlockSpec((D,), lambda i: (0,)),
            pl.BlockSpec((_RB, 1), lambda i: (i, 0)),
        ],
        out_specs=[
            pl.BlockSpec((_RB, D), lambda i: (i, 0)),
            pl.BlockSpec((_RB, 1), lambda i: (i, 0)),
        ],
        out_shape=[
            jax.ShapeDtypeStruct((NPAD, D), jnp.float32),
            jax.ShapeDtypeStruct((NPAD, 1), jnp.float32),
        ],
    )(aggf, aggf, u, dis, b, p, m)


_LO0 = -1080033281
_HI0 = 1069547520


def _sel_body(k, s_ref, m_ref, g_ref):
    s = s_ref[...]
    key = _f32_sort_key(lax.bitcast_convert_type(s, jnp.int32))

    def bis(_, lohi):
        lo, hi = lohi
        mid = (lo >> 1) + (hi >> 1) + (lo & hi & 1)
        cnt = jnp.sum((key >= mid).astype(jnp.int32))
        ok = cnt >= k
        return jnp.where(ok, mid, lo), jnp.where(ok, hi, mid)

    lo, _ = lax.fori_loop(0, 32, bis, (jnp.int32(_LO0), jnp.int32(_HI0)))
    thr = lo
    gt = key > thr
    cnt_gt = jnp.sum(gt.astype(jnp.int32))
    need = k - cnt_gt
    tie = key == thr
    idx = (lax.broadcasted_iota(jnp.int32, (NPAD // 128, 128), 0) * 128
           + lax.broadcasted_iota(jnp.int32, (NPAD // 128, 128), 1))

    def bis2(_, lohi):
        lo2, hi2 = lohi
        mid = (lo2 + hi2) >> 1
        cnt = jnp.sum((tie & (idx < mid)).astype(jnp.int32))
        ok = cnt >= need
        return jnp.where(ok, lo2, mid), jnp.where(ok, mid, hi2)

    _, cut = lax.fori_loop(0, 14, bis2, (jnp.int32(0), jnp.int32(NPAD)))
    m = (gt | (tie & (idx < cut))).astype(jnp.float32)
    m_ref[...] = m
    g_ref[...] = m * s


def _tc_select(sp, k):
    return pl.pallas_call(
        functools.partial(_sel_body, k),
        out_shape=[
            jax.ShapeDtypeStruct((NPAD // 128, 128), jnp.float32),
            jax.ShapeDtypeStruct((NPAD // 128, 128), jnp.float32),
        ],
    )(sp)


def _red_body(h_ref, g_ref, o_ref):
    i = pl.program_id(0)
    part = jnp.sum(h_ref[...] * g_ref[...], axis=0, keepdims=True)

    @pl.when(i == 0)
    def _():
        o_ref[...] = part

    @pl.when(i > 0)
    def _():
        o_ref[...] = o_ref[...] + part


def _tc_reduce(h, g):
    return pl.pallas_call(
        _red_body,
        grid=(_GRID_R,),
        in_specs=[
            pl.BlockSpec((_RB, D), lambda i: (i, 0)),
            pl.BlockSpec((_RB, 1), lambda i: (i, 0)),
        ],
        out_specs=pl.BlockSpec((1, D), lambda i: (0, 0)),
        out_shape=jax.ShapeDtypeStruct((1, D), jnp.float32),
    )(h, g)


def _z_body(x1_ref, x2_ref, w1_ref, b1_ref, w2_ref, b2_ref, wv_ref, bv_ref,
            z_ref, v_ref):
    z0 = x1_ref[...] * (1.0 / K1) + x2_ref[...] * (1.0 / K2)
    z1 = jnp.dot(z0, w1_ref[...], preferred_element_type=jnp.float32) + b1_ref[...][None, :]
    z2 = jnp.dot(z1, w2_ref[...], preferred_element_type=jnp.float32) + b2_ref[...][None, :]
    z_ref[...] = z2
    v_ref[...] = (jnp.dot(z2, wv_ref[...], preferred_element_type=jnp.float32)
                  + bv_ref[...][None, :])


def _tc_z(xs1, xs2, w1, b1, w2, b2, wv, bv):
    return pl.pallas_call(
        _z_body,
        out_shape=[
            jax.ShapeDtypeStruct((1, 64), jnp.float32),
            jax.ShapeDtypeStruct((1, 1), jnp.float32),
        ],
    )(xs1, xs2, w1, b1, w2, b2, wv, bv)


_CB = 2048
_GRID_C = (OUT_DIM + _CB - 1) // _CB


def _out_body(z_ref, w_ref, b_ref, o_ref):
    o_ref[...] = jnp.tanh(
        jnp.dot(z_ref[...], w_ref[...], preferred_element_type=jnp.float32)
        + b_ref[...][None, :])


def _tc_out(z, w3, b3):
    return pl.pallas_call(
        _out_body,
        grid=(_GRID_C,),
        in_specs=[
            pl.BlockSpec((1, 64), lambda i: (0, 0)),
            pl.BlockSpec((64, _CB), lambda i: (0, i)),
            pl.BlockSpec((_CB,), lambda i: (i,)),
        ],
        out_specs=pl.BlockSpec((1, _CB), lambda i: (0, i)),
        out_shape=jax.ShapeDtypeStruct((1, OUT_DIM), jnp.float32),
    )(z, w3, b3)


def kernel(x, edge_index, W1, b1, p1, W2, b2, p2, lin1_W, lin1_b, lin2_W,
           lin2_b, lin3_W, lin3_b, linV_W, linV_b):
    src = edge_index[0]
    dst = edge_index[1]
    xp = jnp.pad(x, ((0, NPAD - N), (0, 0)))
    valid_n = jnp.pad(jnp.ones((N,), jnp.float32), (0, NPAD - N))
    valid_c = valid_n[:, None]
    ones_c = jnp.ones((NPAD, 1), jnp.float32)

    degp1 = _sc_deg(valid_n, src, dst)
    u1, dis1 = _tc_scale(xp, ones_c, W1, degp1)
    aggf1 = _sc_agg(u1, src, dst)
    h, score1 = _tc_post(aggf1, u1, dis1, b1, p1, valid_c)

    m1p, g1p = _tc_select(jnp.reshape(score1, (NPAD // 128, 128)), K1)
    mask1 = jnp.reshape(m1p, (NPAD,))
    g1 = jnp.reshape(g1p, (NPAD, 1))
    xs1 = _tc_reduce(h, g1)

    degp2 = _sc_deg(mask1, src, dst)
    u2, dis2 = _tc_scale(h, g1, W2, degp2)
    aggf2 = _sc_agg(u2, src, dst)
    h2, score2 = _tc_post(aggf2, u2, dis2, b2, p2, mask1[:, None])

    _, g2p = _tc_select(jnp.reshape(score2, (NPAD // 128, 128)), K2)
    g2 = jnp.reshape(g2p, (NPAD, 1))
    xs2 = _tc_reduce(h2, g2)

    z, value = _tc_z(xs1, xs2, lin1_W, lin1_b, lin2_W, lin2_b, linV_W, linV_b)
    out = _tc_out(z, lin3_W, lin3_b)
    return (out, value)


# speedup vs baseline: 21.8083x; 1.1511x over previous
import functools

import jax
import jax.numpy as jnp
from jax import lax
from jax.experimental import pallas as pl
from jax.experimental.pallas import tpu as pltpu
from jax.experimental.pallas import tpu_sc as plsc

N = 10000
E = 320000
D = 128
K1 = 8000
K2 = 6400
OUT_DIM = 40000
NPAD = 10240

NC = 2
NS = 16
NW = NC * NS
EPT = E // NW
CHUNK = 80
NCHUNK = EPT // CHUNK
RPT = NPAD // NS
DW = 16


def _f32_sort_key(bits):
    return jnp.where(bits < 0, (~bits) ^ (-2147483648), bits)


def _sc_mesh():
    return plsc.VectorSubcoreMesh(
        core_axis_name="c", subcore_axis_name="s", num_cores=NC, num_subcores=NS)


def _agg_body(w, u_hbm, src_hbm, dst_hbm, out_hbm, sidx_v, didx_v, rows_v,
              shared, sem):
    c = lax.axis_index("c")
    s = lax.axis_index("s")
    wid = c * NS + s
    base = wid * EPT

    def zrow(i, carry):
        def zcol(j, carry2):
            rows_v[i, pl.ds(j * 16, 16)] = jnp.zeros((16,), jnp.float32)
            return carry2
        return lax.fori_loop(0, w // 16, zcol, carry)

    lax.fori_loop(0, CHUNK, zrow, 0)
    off = s * RPT
    for t in range(RPT // CHUNK):
        pltpu.sync_copy(rows_v, shared.at[pl.ds(off + t * CHUNK, CHUNK)])
    plsc.subcore_barrier()

    def chunk(i, carry):
        cb = base + i * CHUNK
        pltpu.sync_copy(src_hbm.at[pl.ds(cb, CHUNK)], sidx_v)
        pltpu.sync_copy(dst_hbm.at[pl.ds(cb, CHUNK)], didx_v)
        pltpu.async_copy(u_hbm.at[sidx_v], rows_v, sem).wait()
        pltpu.sync_copy(rows_v, shared.at[didx_v], add=True)
        return carry

    lax.fori_loop(0, NCHUNK, chunk, 0)
    plsc.subcore_barrier()
    pltpu.sync_copy(shared.at[pl.ds(off, RPT)],
                    out_hbm.at[pl.ds(c * NPAD + off, RPT)])


def _sc_agg(u, src, dst):
    w = u.shape[1]
    f = pl.kernel(
        functools.partial(_agg_body, w),
        out_type=jax.ShapeDtypeStruct((2 * NPAD, w), jnp.float32),
        mesh=_sc_mesh(),
        scratch_types=[
            pltpu.VMEM((CHUNK,), jnp.int32),
            pltpu.VMEM((CHUNK,), jnp.int32),
            pltpu.VMEM((CHUNK, w), jnp.float32),
            pltpu.VMEM_SHARED((NPAD, w), jnp.float32),
            pltpu.SemaphoreType.DMA,
        ],
        compiler_params=pltpu.CompilerParams(use_tc_tiling_on_sc=(w == D)),
    )
    return f(u, src, dst)


def _sc_deg(mask_pad, src, dst):
    table = jnp.broadcast_to(mask_pad[:, None], (NPAD, DW))
    return _sc_agg(table, src, dst)


_RB = 1024
_GRID_R = NPAD // _RB


def _scale_body(x_ref, rs_ref, w_ref, d0_ref, d1_ref, u_ref, dis_ref):
    xw = jnp.dot(x_ref[...] * rs_ref[...], w_ref[...],
                 preferred_element_type=jnp.float32)
    deg = d0_ref[...][:, 0:1] + d1_ref[...][:, 0:1] + 1.0
    dis = lax.rsqrt(jnp.maximum(deg, 1.0))
    u_ref[...] = xw * dis
    dis_ref[...] = dis


def _tc_scale(x, rs, w, degp):
    return pl.pallas_call(
        _scale_body,
        grid=(_GRID_R,),
        in_specs=[
            pl.BlockSpec((_RB, D), lambda i: (i, 0)),
            pl.BlockSpec((_RB, 1), lambda i: (i, 0)),
            pl.BlockSpec((D, D), lambda i: (0, 0)),
            pl.BlockSpec((_RB, DW), lambda i: (i, 0)),
            pl.BlockSpec((_RB, DW), lambda i: (i + _GRID_R, 0)),
        ],
        out_specs=[
            pl.BlockSpec((_RB, D), lambda i: (i, 0)),
            pl.BlockSpec((_RB, 1), lambda i: (i, 0)),
        ],
        out_shape=[
            jax.ShapeDtypeStruct((NPAD, D), jnp.float32),
            jax.ShapeDtypeStruct((NPAD, 1), jnp.float32),
        ],
    )(x, rs, w, degp, degp)


def _post_body(a0_ref, a1_ref, u_ref, dis_ref, b_ref, p_ref, m_ref, h_ref, sc_ref):
    t = (a0_ref[...] + a1_ref[...] + u_ref[...]) * dis_ref[...] + b_ref[...][None, :]
    h = jnp.maximum(t, 0.0)
    h_ref[...] = h
    pv = p_ref[...]
    inv = lax.rsqrt(jnp.sum(pv * pv))
    s = jnp.sum(h * pv[None, :], axis=1, keepdims=True) * inv
    sc = jnp.tanh(s)
    sc_ref[...] = jnp.where(m_ref[...] > 0, sc, -2.0)


def _tc_post(aggf, u, dis, b, p, m):
    return pl.pallas_call(
        _post_body,
        grid=(_GRID_R,),
        in_specs=[
            pl.BlockSpec((_RB, D), lambda i: (i, 0)),
            pl.BlockSpec((_RB, D), lambda i: (i + _GRID_R, 0)),
            pl.BlockSpec((_RB, D), lambda i: (i, 0)),
            pl.BlockSpec((_RB, 1), lambda i: (i, 0)),
            pl.BlockSpec((D,), lambda i: (0,)),
            pl.BlockSpec((D,), lambda i: (0,)),
            pl.BlockSpec((_RB, 1), lambda i: (i, 0)),
        ],
        out_specs=[
            pl.BlockSpec((_RB, D), lambda i: (i, 0)),
            pl.BlockSpec((_RB, 1), lambda i: (i, 0)),
        ],
        out_shape=[
            jax.ShapeDtypeStruct((NPAD, D), jnp.float32),
            jax.ShapeDtypeStruct((NPAD, 1), jnp.float32),
        ],
    )(aggf, aggf, u, dis, b, p, m)


_LO0 = -1080033281
_HI0 = 1069547520


def _sel_body(k, s_ref, m_ref, g_ref):
    s = s_ref[...]
    key = _f32_sort_key(lax.bitcast_convert_type(s, jnp.int32))

    def bis(_, lohi):
        lo, hi = lohi
        mid = (lo >> 1) + (hi >> 1) + (lo & hi & 1)
        cnt = jnp.sum((key >= mid).astype(jnp.int32))
        ok = cnt >= k
        return jnp.where(ok, mid, lo), jnp.where(ok, hi, mid)

    lo, _ = lax.fori_loop(0, 32, bis, (jnp.int32(_LO0), jnp.int32(_HI0)))
    thr = lo
    gt = key > thr
    cnt_gt = jnp.sum(gt.astype(jnp.int32))
    need = k - cnt_gt
    tie = key == thr
    idx = (lax.broadcasted_iota(jnp.int32, (NPAD // 128, 128), 0) * 128
           + lax.broadcasted_iota(jnp.int32, (NPAD // 128, 128), 1))

    def bis2(_, lohi):
        lo2, hi2 = lohi
        mid = (lo2 + hi2) >> 1
        cnt = jnp.sum((tie & (idx < mid)).astype(jnp.int32))
        ok = cnt >= need
        return jnp.where(ok, lo2, mid), jnp.where(ok, mid, hi2)

    _, cut = lax.fori_loop(0, 14, bis2, (jnp.int32(0), jnp.int32(NPAD)))
    m = (gt | (tie & (idx < cut))).astype(jnp.float32)
    m_ref[...] = m
    g_ref[...] = m * s


def _tc_select(sp, k):
    return pl.pallas_call(
        functools.partial(_sel_body, k),
        out_shape=[
            jax.ShapeDtypeStruct((NPAD // 128, 128), jnp.float32),
            jax.ShapeDtypeStruct((NPAD // 128, 128), jnp.float32),
        ],
    )(sp)


def _red_body(h_ref, g_ref, o_ref):
    i = pl.program_id(0)
    part = jnp.sum(h_ref[...] * g_ref[...], axis=0, keepdims=True)

    @pl.when(i == 0)
    def _():
        o_ref[...] = part

    @pl.when(i > 0)
    def _():
        o_ref[...] = o_ref[...] + part


def _tc_reduce(h, g):
    return pl.pallas_call(
        _red_body,
        grid=(_GRID_R,),
        in_specs=[
            pl.BlockSpec((_RB, D), lambda i: (i, 0)),
            pl.BlockSpec((_RB, 1), lambda i: (i, 0)),
        ],
        out_specs=pl.BlockSpec((1, D), lambda i: (0, 0)),
        out_shape=jax.ShapeDtypeStruct((1, D), jnp.float32),
    )(h, g)


def _z_body(x1_ref, x2_ref, w1_ref, b1_ref, w2_ref, b2_ref, wv_ref, bv_ref,
            z_ref, v_ref):
    z0 = x1_ref[...] * (1.0 / K1) + x2_ref[...] * (1.0 / K2)
    z1 = jnp.dot(z0, w1_ref[...], preferred_element_type=jnp.float32) + b1_ref[...][None, :]
    z2 = jnp.dot(z1, w2_ref[...], preferred_element_type=jnp.float32) + b2_ref[...][None, :]
    z_ref[...] = z2
    v_ref[...] = (jnp.dot(z2, wv_ref[...], preferred_element_type=jnp.float32)
                  + bv_ref[...][None, :])


def _tc_z(xs1, xs2, w1, b1, w2, b2, wv, bv):
    return pl.pallas_call(
        _z_body,
        out_shape=[
            jax.ShapeDtypeStruct((1, 64), jnp.float32),
            jax.ShapeDtypeStruct((1, 1), jnp.float32),
        ],
    )(xs1, xs2, w1, b1, w2, b2, wv, bv)


_CB = 2048
_GRID_C = (OUT_DIM + _CB - 1) // _CB


def _out_body(z_ref, w_ref, b_ref, o_ref):
    o_ref[...] = jnp.tanh(
        jnp.dot(z_ref[...], w_ref[...], preferred_element_type=jnp.float32)
        + b_ref[...][None, :])


def _tc_out(z, w3, b3):
    return pl.pallas_call(
        _out_body,
        grid=(_GRID_C,),
        in_specs=[
            pl.BlockSpec((1, 64), lambda i: (0, 0)),
            pl.BlockSpec((64, _CB), lambda i: (0, i)),
            pl.BlockSpec((_CB,), lambda i: (i,)),
        ],
        out_specs=pl.BlockSpec((1, _CB), lambda i: (0, i)),
        out_shape=jax.ShapeDtypeStruct((1, OUT_DIM), jnp.float32),
    )(z, w3, b3)


def kernel(x, edge_index, W1, b1, p1, W2, b2, p2, lin1_W, lin1_b, lin2_W,
           lin2_b, lin3_W, lin3_b, linV_W, linV_b):
    src = edge_index[0]
    dst = edge_index[1]
    xp = jnp.pad(x, ((0, NPAD - N), (0, 0)))
    valid_n = jnp.pad(jnp.ones((N,), jnp.float32), (0, NPAD - N))
    valid_c = valid_n[:, None]
    ones_c = jnp.ones((NPAD, 1), jnp.float32)

    degp1 = _sc_deg(valid_n, src, dst)
    u1, dis1 = _tc_scale(xp, ones_c, W1, degp1)
    aggf1 = _sc_agg(u1, src, dst)
    h, score1 = _tc_post(aggf1, u1, dis1, b1, p1, valid_c)

    m1p, g1p = _tc_select(jnp.reshape(score1, (NPAD // 128, 128)), K1)
    mask1 = jnp.reshape(m1p, (NPAD,))
    g1 = jnp.reshape(g1p, (NPAD, 1))
    xs1 = _tc_reduce(h, g1)

    degp2 = _sc_deg(mask1, src, dst)
    u2, dis2 = _tc_scale(h, g1, W2, degp2)
    aggf2 = _sc_agg(u2, src, dst)
    h2, score2 = _tc_post(aggf2, u2, dis2, b2, p2, mask1[:, None])

    _, g2p = _tc_select(jnp.reshape(score2, (NPAD // 128, 128)), K2)
    g2 = jnp.reshape(g2p, (NPAD, 1))
    xs2 = _tc_reduce(h2, g2)

    z, value = _tc_z(xs1, xs2, lin1_W, lin1_b, lin2_W, lin2_b, linV_W, linV_b)
    out = _tc_out(z, lin3_W, lin3_b)
    return (out, value)


# speedup vs baseline: 33.9993x; 1.5590x over previous
import functools

import jax
import jax.numpy as jnp
from jax import lax
from jax.experimental import pallas as pl
from jax.experimental.pallas import tpu as pltpu
from jax.experimental.pallas import tpu_sc as plsc

N = 10000
E = 320000
D = 128
K1 = 8000
K2 = 6400
OUT_DIM = 40000
NPAD = 10240

NC = 2
NS = 16
NW = NC * NS
CHUNK = 96
NCHUNK_D = 106
EPT_D = NCHUNK_D * CHUNK
EPAD_D = NW * EPT_D
NCHUNK_A = 210
EPT_A = NCHUNK_A * CHUNK
EPAD_A = NS * EPT_A
DH = 64
RPT = NPAD // NS
DW = 16
SRC_PAD = N
DST_PAD = N + 1


def _f32_sort_key(bits):
    return jnp.where(bits < 0, (~bits) ^ (-2147483648), bits)


def _sc_mesh():
    return plsc.VectorSubcoreMesh(
        core_axis_name="c", subcore_axis_name="s", num_cores=NC, num_subcores=NS)


def _agg_body(w, nchunk, u_hbm, src_hbm, dst_hbm, out_hbm, sidx_v, didx_v,
              buf0, buf1, shared, sem0, sem1):
    c = lax.axis_index("c")
    s = lax.axis_index("s")
    wid = c * NS + s

    pltpu.sync_copy(src_hbm.at[wid], sidx_v)
    pltpu.sync_copy(dst_hbm.at[wid], didx_v)

    def zrow(i, carry):
        def zcol(j, carry2):
            buf0[i, pl.ds(j * 16, 16)] = jnp.zeros((16,), jnp.float32)
            return carry2
        return lax.fori_loop(0, w // 16, zcol, carry)

    lax.fori_loop(0, CHUNK, zrow, 0)
    off = s * RPT
    for t in range(RPT // CHUNK):
        pltpu.sync_copy(buf0, shared.at[pl.ds(off + t * CHUNK, CHUNK)])
    rem = RPT - (RPT // CHUNK) * CHUNK
    pltpu.sync_copy(buf0.at[pl.ds(0, rem)],
                    shared.at[pl.ds(off + (RPT // CHUNK) * CHUNK, rem)])
    plsc.subcore_barrier()

    pltpu.async_copy(u_hbm.at[sidx_v.at[0]], buf0, sem0)

    def pair(j, carry):
        i0 = 2 * j
        pltpu.async_copy(u_hbm.at[sidx_v.at[i0 + 1]], buf1, sem1)
        pltpu.make_async_copy(u_hbm.at[sidx_v.at[i0]], buf0, sem0).wait()
        pltpu.sync_copy(buf0, shared.at[didx_v.at[i0]], add=True)

        @pl.when(j < nchunk // 2 - 1)
        def _():
            pltpu.async_copy(u_hbm.at[sidx_v.at[i0 + 2]], buf0, sem0)

        pltpu.make_async_copy(u_hbm.at[sidx_v.at[i0 + 1]], buf1, sem1).wait()
        pltpu.sync_copy(buf1, shared.at[didx_v.at[i0 + 1]], add=True)
        return carry

    lax.fori_loop(0, nchunk // 2, pair, 0)
    plsc.subcore_barrier()
    pltpu.sync_copy(shared.at[pl.ds(off, RPT)],
                    out_hbm.at[pl.ds(c * NPAD + off, RPT)])


def _sc_agg(u, src3, dst3):
    w = u.shape[1]
    nchunk = src3.shape[1]
    f = pl.kernel(
        functools.partial(_agg_body, w, nchunk),
        out_type=jax.ShapeDtypeStruct((2 * NPAD, w), jnp.float32),
        mesh=_sc_mesh(),
        scratch_types=[
            pltpu.VMEM((nchunk, CHUNK), jnp.int32),
            pltpu.VMEM((nchunk, CHUNK), jnp.int32),
            pltpu.VMEM((CHUNK, w), jnp.float32),
            pltpu.VMEM((CHUNK, w), jnp.float32),
            pltpu.VMEM_SHARED((NPAD, w), jnp.float32),
            pltpu.SemaphoreType.DMA,
            pltpu.SemaphoreType.DMA,
        ],
        compiler_params=pltpu.CompilerParams(use_tc_tiling_on_sc=False),
    )
    return f(u, src3, dst3)


def _sc_deg(mask_pad, src3, dst3):
    table = jnp.broadcast_to(mask_pad[:, None], (NPAD, DW))
    return _sc_agg(table, src3, dst3)


_RB = 1024
_GRID_R = NPAD // _RB


def _scale_body(x_ref, rs_ref, w_ref, d0_ref, d1_ref, u_ref, dis_ref):
    xw = jnp.dot(x_ref[...] * rs_ref[...], w_ref[...],
                 preferred_element_type=jnp.float32)
    deg = d0_ref[...][:, 0:1] + d1_ref[...][:, 0:1] + 1.0
    dis = lax.rsqrt(jnp.maximum(deg, 1.0))
    u_ref[...] = xw * dis
    dis_ref[...] = dis


def _tc_scale(x, rs, w, degp):
    return pl.pallas_call(
        _scale_body,
        grid=(_GRID_R,),
        in_specs=[
            pl.BlockSpec((_RB, D), lambda i: (i, 0)),
            pl.BlockSpec((_RB, 1), lambda i: (i, 0)),
            pl.BlockSpec((D, D), lambda i: (0, 0)),
            pl.BlockSpec((_RB, DW), lambda i: (i, 0)),
            pl.BlockSpec((_RB, DW), lambda i: (i + _GRID_R, 0)),
        ],
        out_specs=[
            pl.BlockSpec((_RB, D), lambda i: (i, 0)),
            pl.BlockSpec((_RB, 1), lambda i: (i, 0)),
        ],
        out_shape=[
            jax.ShapeDtypeStruct((NPAD, D), jnp.float32),
            jax.ShapeDtypeStruct((NPAD, 1), jnp.float32),
        ],
    )(x, rs, w, degp, degp)


def _post_body(alo_ref, ahi_ref, u_ref, dis_ref, b_ref, p_ref, m_ref, h_ref, sc_ref):
    u = u_ref[...]
    dis = dis_ref[...]
    b = b_ref[...]
    h_lo = jnp.maximum(
        (alo_ref[...] + u[:, :DH]) * dis + b[None, :DH], 0.0)
    h_hi = jnp.maximum(
        (ahi_ref[...] + u[:, DH:]) * dis + b[None, DH:], 0.0)
    h_ref[:, :DH] = h_lo
    h_ref[:, DH:] = h_hi
    pv = p_ref[...]
    inv = lax.rsqrt(jnp.sum(pv * pv))
    s = (jnp.sum(h_lo * pv[None, :DH], axis=1, keepdims=True)
         + jnp.sum(h_hi * pv[None, DH:], axis=1, keepdims=True)) * inv
    sc = jnp.tanh(s)
    sc_ref[...] = jnp.where(m_ref[...] > 0, sc, -2.0)


def _tc_post(aggf, u, dis, b, p, m):
    return pl.pallas_call(
        _post_body,
        grid=(_GRID_R,),
        in_specs=[
            pl.BlockSpec((_RB, DH), lambda i: (i, 0)),
            pl.BlockSpec((_RB, DH), lambda i: (i + _GRID_R, 0)),
            pl.BlockSpec((_RB, D), lambda i: (i, 0)),
            pl.BlockSpec((_RB, 1), lambda i: (i, 0)),
            pl.BlockSpec((D,), lambda i: (0,)),
            pl.BlockSpec((D,), lambda i: (0,)),
            pl.BlockSpec((_RB, 1), lambda i: (i, 0)),
        ],
        out_specs=[
            pl.BlockSpec((_RB, D), lambda i: (i, 0)),
            pl.BlockSpec((_RB, 1), lambda i: (i, 0)),
        ],
        out_shape=[
            jax.ShapeDtypeStruct((NPAD, D), jnp.float32),
            jax.ShapeDtypeStruct((NPAD, 1), jnp.float32),
        ],
    )(aggf, aggf, u, dis, b, p, m)


_LO0 = -1080033281
_HI0 = 1069547520


def _sel_body(k, s_ref, m_ref, g_ref):
    s = s_ref[...]
    key = _f32_sort_key(lax.bitcast_convert_type(s, jnp.int32))

    def bis(_, lohi):
        lo, hi = lohi
        mid = (lo >> 1) + (hi >> 1) + (lo & hi & 1)
        cnt = jnp.sum((key >= mid).astype(jnp.int32))
        ok = cnt >= k
        return jnp.where(ok, mid, lo), jnp.where(ok, hi, mid)

    lo, _ = lax.fori_loop(0, 32, bis, (jnp.int32(_LO0), jnp.int32(_HI0)))
    thr = lo
    gt = key > thr
    cnt_gt = jnp.sum(gt.astype(jnp.int32))
    need = k - cnt_gt
    tie = key == thr
    idx = (lax.broadcasted_iota(jnp.int32, (NPAD // 128, 128), 0) * 128
           + lax.broadcasted_iota(jnp.int32, (NPAD // 128, 128), 1))

    def bis2(_, lohi):
        lo2, hi2 = lohi
        mid = (lo2 + hi2) >> 1
        cnt = jnp.sum((tie & (idx < mid)).astype(jnp.int32))
        ok = cnt >= need
        return jnp.where(ok, lo2, mid), jnp.where(ok, mid, hi2)

    _, cut = lax.fori_loop(0, 14, bis2, (jnp.int32(0), jnp.int32(NPAD)))
    m = (gt | (tie & (idx < cut))).astype(jnp.float32)
    m_ref[...] = m
    g_ref[...] = m * s


def _tc_select(sp, k):
    return pl.pallas_call(
        functools.partial(_sel_body, k),
        out_shape=[
            jax.ShapeDtypeStruct((NPAD // 128, 128), jnp.float32),
            jax.ShapeDtypeStruct((NPAD // 128, 128), jnp.float32),
        ],
    )(sp)


def _red_body(h_ref, g_ref, o_ref):
    i = pl.program_id(0)
    part = jnp.sum(h_ref[...] * g_ref[...], axis=0, keepdims=True)

    @pl.when(i == 0)
    def _():
        o_ref[...] = part

    @pl.when(i > 0)
    def _():
        o_ref[...] = o_ref[...] + part


def _tc_reduce(h, g):
    return pl.pallas_call(
        _red_body,
        grid=(_GRID_R,),
        in_specs=[
            pl.BlockSpec((_RB, D), lambda i: (i, 0)),
            pl.BlockSpec((_RB, 1), lambda i: (i, 0)),
        ],
        out_specs=pl.BlockSpec((1, D), lambda i: (0, 0)),
        out_shape=jax.ShapeDtypeStruct((1, D), jnp.float32),
    )(h, g)


def _z_body(x1_ref, x2_ref, w1_ref, b1_ref, w2_ref, b2_ref, wv_ref, bv_ref,
            z_ref, v_ref):
    z0 = x1_ref[...] * (1.0 / K1) + x2_ref[...] * (1.0 / K2)
    z1 = jnp.dot(z0, w1_ref[...], preferred_element_type=jnp.float32) + b1_ref[...][None, :]
    z2 = jnp.dot(z1, w2_ref[...], preferred_element_type=jnp.float32) + b2_ref[...][None, :]
    z_ref[...] = z2
    v_ref[...] = (jnp.dot(z2, wv_ref[...], preferred_element_type=jnp.float32)
                  + bv_ref[...][None, :])


def _tc_z(xs1, xs2, w1, b1, w2, b2, wv, bv):
    return pl.pallas_call(
        _z_body,
        out_shape=[
            jax.ShapeDtypeStruct((1, 64), jnp.float32),
            jax.ShapeDtypeStruct((1, 1), jnp.float32),
        ],
    )(xs1, xs2, w1, b1, w2, b2, wv, bv)


_CB = 2048
_GRID_C = (OUT_DIM + _CB - 1) // _CB


def _out_body(z_ref, w_ref, b_ref, o_ref):
    o_ref[...] = jnp.tanh(
        jnp.dot(z_ref[...], w_ref[...], preferred_element_type=jnp.float32)
        + b_ref[...][None, :])


def _tc_out(z, w3, b3):
    return pl.pallas_call(
        _out_body,
        grid=(_GRID_C,),
        in_specs=[
            pl.BlockSpec((1, 64), lambda i: (0, 0)),
            pl.BlockSpec((64, _CB), lambda i: (0, i)),
            pl.BlockSpec((_CB,), lambda i: (i,)),
        ],
        out_specs=pl.BlockSpec((1, _CB), lambda i: (0, i)),
        out_shape=jax.ShapeDtypeStruct((1, OUT_DIM), jnp.float32),
    )(z, w3, b3)


def kernel(x, edge_index, W1, b1, p1, W2, b2, p2, lin1_W, lin1_b, lin2_W,
           lin2_b, lin3_W, lin3_b, linV_W, linV_b):
    src_d = jnp.reshape(
        jnp.concatenate(
            [edge_index[0], jnp.full((EPAD_D - E,), SRC_PAD, jnp.int32)]),
        (NW, NCHUNK_D, CHUNK))
    dst_d = jnp.reshape(
        jnp.concatenate(
            [edge_index[1], jnp.full((EPAD_D - E,), DST_PAD, jnp.int32)]),
        (NW, NCHUNK_D, CHUNK))
    src_a0 = jnp.reshape(
        jnp.concatenate(
            [edge_index[0], jnp.full((EPAD_A - E,), SRC_PAD, jnp.int32)]),
        (NS, NCHUNK_A, CHUNK))
    src_a = jnp.concatenate([src_a0, src_a0 + NPAD], axis=0)
    dst_a0 = jnp.reshape(
        jnp.concatenate(
            [edge_index[1], jnp.full((EPAD_A - E,), DST_PAD, jnp.int32)]),
        (NS, NCHUNK_A, CHUNK))
    dst_a = jnp.concatenate([dst_a0, dst_a0], axis=0)

    xp = jnp.pad(x, ((0, NPAD - N), (0, 0)))
    valid_n = jnp.pad(jnp.ones((N,), jnp.float32), (0, NPAD - N))
    valid_c = valid_n[:, None]
    ones_c = jnp.ones((NPAD, 1), jnp.float32)

    def lane_split(u):
        return jnp.concatenate([u[:, :DH], u[:, DH:]], axis=0)

    degp1 = _sc_deg(valid_n, src_d, dst_d)
    u1, dis1 = _tc_scale(xp, ones_c, W1, degp1)
    aggf1 = _sc_agg(lane_split(u1), src_a, dst_a)
    h, score1 = _tc_post(aggf1, u1, dis1, b1, p1, valid_c)

    m1p, g1p = _tc_select(jnp.reshape(score1, (NPAD // 128, 128)), K1)
    mask1 = jnp.reshape(m1p, (NPAD,))
    g1 = jnp.reshape(g1p, (NPAD, 1))
    xs1 = _tc_reduce(h, g1)

    degp2 = _sc_deg(mask1, src_d, dst_d)
    u2, dis2 = _tc_scale(h, g1, W2, degp2)
    aggf2 = _sc_agg(lane_split(u2), src_a, dst_a)
    h2, score2 = _tc_post(aggf2, u2, dis2, b2, p2, mask1[:, None])

    _, g2p = _tc_select(jnp.reshape(score2, (NPAD // 128, 128)), K2)
    g2 = jnp.reshape(g2p, (NPAD, 1))
    xs2 = _tc_reduce(h2, g2)

    z, value = _tc_z(xs1, xs2, lin1_W, lin1_b, lin2_W, lin2_b, linV_W, linV_b)
    out = _tc_out(z, lin3_W, lin3_b)
    return (out, value)
